# baseline (speedup 1.0000x reference)
import functools

import jax
import jax.numpy as jnp
from jax import lax
from jax.experimental import pallas as pl
from jax.experimental.pallas import tpu as pltpu
from jax.experimental.pallas import tpu_sc as plsc

_DEPTH = 3
_NC = 2
_NS = 16
_NW = _NC * _NS
_CW = 80
_SW = 40
_NBUF = 5


def _row_block(bs, d):
    return pl.BlockSpec((bs, d), lambda i: (i, 0))


def _off_block(bs, d, blk_off):
    return pl.BlockSpec((bs, d), lambda i, o=blk_off: (o + i, 0))


def _rep_block(shape):
    return pl.BlockSpec(shape, lambda i: tuple(0 for _ in shape))


def _sc_gather1h(table, idx, off, eh):
    d = table.shape[1]
    per_w = eh // _NW
    n_ch = per_w // _CW
    n_gr = n_ch // _NBUF
    mesh = plsc.VectorSubcoreMesh(core_axis_name="c", subcore_axis_name="s")

    @functools.partial(
        pl.kernel,
        out_type=jax.ShapeDtypeStruct((eh, d), jnp.float32),
        mesh=mesh,
        scratch_types=[
            pltpu.VMEM((_NBUF, _CW), jnp.int32),
            pltpu.VMEM((_NBUF, _CW, d), jnp.float32),
            pltpu.SemaphoreType.DMA((_NBUF,)),
            pltpu.SemaphoreType.DMA((_NBUF,)),
            pltpu.SemaphoreType.DMA((_NBUF,)),
        ],
    )
    def gk(table_hbm, idx_hbm, out_hbm, idx_v, rows_v, isem, gsem, ssem):
        wid = lax.axis_index("s") * _NC + lax.axis_index("c")
        base = wid * per_w

        def idx_copy(c, b):
            return pltpu.make_async_copy(
                idx_hbm.at[pl.ds(off + base + c * _CW, _CW)], idx_v.at[b],
                isem.at[b])

        def out_copy(c, b):
            return pltpu.make_async_copy(
                rows_v.at[b], out_hbm.at[pl.ds(base + c * _CW, _CW)],
                ssem.at[b])

        for b in range(_NBUF):
            idx_copy(b, b).start()

        def group(g, carry):
            c0 = g * _NBUF
            gathers = []
            for b in range(_NBUF):
                @pl.when(g > 0)
                def _():
                    out_copy(0, b).wait()

                idx_copy(c0 + b, b).wait()
                gathers.append(pltpu.async_copy(
                    table_hbm.at[idx_v.at[b]], rows_v.at[b], gsem.at[b]))
            for b in range(_NBUF):
                gathers[b].wait()
                out_copy(c0 + b, b).start()

                @pl.when(g + 1 < n_gr)
                def _():
                    idx_copy(c0 + _NBUF + b, b).start()
            return carry

        lax.fori_loop(0, n_gr, group, 0)
        for b in range(_NBUF):
            out_copy(0, b).wait()

    return gk(table, idx)


def _sc_gather2h(table_a, idx_a, table_b, idx_b, off, eh):
    d = table_a.shape[1]
    per_w = eh // _NW
    n_ch = per_w // _CW
    n_gr = n_ch // _NBUF
    mesh = plsc.VectorSubcoreMesh(core_axis_name="c", subcore_axis_name="s")

    @functools.partial(
        pl.kernel,
        out_type=(jax.ShapeDtypeStruct((eh, d), jnp.float32),
                  jax.ShapeDtypeStruct((eh, d), jnp.float32)),
        mesh=mesh,
        scratch_types=[
            pltpu.VMEM((_NBUF, _CW), jnp.int32),
            pltpu.VMEM((_NBUF, _CW), jnp.int32),
            pltpu.VMEM((_NBUF, _CW, d), jnp.float32),
            pltpu.VMEM((_NBUF, _CW, d), jnp.float32),
            pltpu.SemaphoreType.DMA((_NBUF,)),
            pltpu.SemaphoreType.DMA((_NBUF,)),
            pltpu.SemaphoreType.DMA((_NBUF,)),
            pltpu.SemaphoreType.DMA((_NBUF,)),
            pltpu.SemaphoreType.DMA((_NBUF,)),
            pltpu.SemaphoreType.DMA((_NBUF,)),
        ],
    )
    def gk(ta_hbm, ia_hbm, tb_hbm, ib_hbm, outa_hbm, outb_hbm,
           ia_v, ib_v, ra_v, rb_v, isema, isemb, gsema, gsemb, ssema, ssemb):
        wid = lax.axis_index("s") * _NC + lax.axis_index("c")
        base = wid * per_w

        def icopy(hbm, vref, sem, c, b):
            return pltpu.make_async_copy(
                hbm.at[pl.ds(off + base + c * _CW, _CW)], vref.at[b],
                sem.at[b])

        def ocopy(vref, hbm, sem, c, b):
            return pltpu.make_async_copy(
                vref.at[b], hbm.at[pl.ds(base + c * _CW, _CW)], sem.at[b])

        for b in range(_NBUF):
            icopy(ia_hbm, ia_v, isema, b, b).start()
            icopy(ib_hbm, ib_v, isemb, b, b).start()

        def group(g, carry):
            c0 = g * _NBUF
            gathers = []
            for b in range(_NBUF):
                @pl.when(g > 0)
                def _():
                    ocopy(ra_v, outa_hbm, ssema, 0, b).wait()
                    ocopy(rb_v, outb_hbm, ssemb, 0, b).wait()

                icopy(ia_hbm, ia_v, isema, c0 + b, b).wait()
                icopy(ib_hbm, ib_v, isemb, c0 + b, b).wait()
                gathers.append((
                    pltpu.async_copy(ta_hbm.at[ia_v.at[b]], ra_v.at[b],
                                     gsema.at[b]),
                    pltpu.async_copy(tb_hbm.at[ib_v.at[b]], rb_v.at[b],
                                     gsemb.at[b])))
            for b in range(_NBUF):
                ga, gb = gathers[b]
                ga.wait()
                ocopy(ra_v, outa_hbm, ssema, c0 + b, b).start()
                gb.wait()
                ocopy(rb_v, outb_hbm, ssemb, c0 + b, b).start()

                @pl.when(g + 1 < n_gr)
                def _():
                    icopy(ia_hbm, ia_v, isema, c0 + _NBUF + b, b).start()
                    icopy(ib_hbm, ib_v, isemb, c0 + _NBUF + b, b).start()
            return carry

        lax.fori_loop(0, n_gr, group, 0)
        for b in range(_NBUF):
            ocopy(ra_v, outa_hbm, ssema, 0, b).wait()
            ocopy(rb_v, outb_hbm, ssemb, 0, b).wait()

    return gk(table_a, idx_a, table_b, idx_b)


def _sc_scatter_h(vals, idx, off, zeros_nd):
    eh, d = vals.shape
    n = zeros_nd.shape[0]
    per_w = eh // _NW
    n_ch = per_w // _SW
    n_gr = n_ch // _NBUF
    rpt = 640
    tail = n - (_NS - 1) * rpt
    mesh = plsc.VectorSubcoreMesh(core_axis_name="c", subcore_axis_name="s")

    @functools.partial(
        pl.kernel,
        out_type=jax.ShapeDtypeStruct((_NC, n, d), jnp.float32),
        mesh=mesh,
        scratch_types=[
            pltpu.VMEM((_NBUF, _SW), jnp.int32),
            pltpu.VMEM((_NBUF, _SW, d), jnp.float32),
            pltpu.VMEM_SHARED((n, d), jnp.float32),
            pltpu.SemaphoreType.DMA((_NBUF,)),
            pltpu.SemaphoreType.DMA((_NBUF,)),
            pltpu.SemaphoreType.DMA((_NBUF,)),
        ],
    )
    def sk(vals_hbm, idx_hbm, zeros_hbm, out_hbm, idx_v, rows_v, acc_sh,
           isem, vsem, ssem):
        cid = lax.axis_index("c")
        sid = lax.axis_index("s")
        wid = sid * _NC + cid
        base = wid * per_w

        @pl.when(sid < _NS - 1)
        def _():
            pltpu.sync_copy(zeros_hbm.at[pl.ds(sid * rpt, rpt)],
                            acc_sh.at[pl.ds(sid * rpt, rpt)])

        @pl.when(sid == _NS - 1)
        def _():
            pltpu.sync_copy(zeros_hbm.at[pl.ds((_NS - 1) * rpt, tail)],
                            acc_sh.at[pl.ds((_NS - 1) * rpt, tail)])

        plsc.subcore_barrier()

        def idx_copy(c, b):
            return pltpu.make_async_copy(
                idx_hbm.at[pl.ds(off + base + c * _SW, _SW)], idx_v.at[b],
                isem.at[b])

        def val_copy(c, b):
            return pltpu.make_async_copy(
                vals_hbm.at[pl.ds(base + c * _SW, _SW)], rows_v.at[b],
                vsem.at[b])

        for b in range(_NBUF):
            idx_copy(b, b).start()
            val_copy(b, b).start()

        def group(g, carry):
            c0 = g * _NBUF
            scat = []
            for b in range(_NBUF):
                idx_copy(c0 + b, b).wait()
                val_copy(c0 + b, b).wait()
                scat.append(pltpu.async_copy(
                    rows_v.at[b], acc_sh.at[idx_v.at[b]], ssem.at[b],
                    add=True))
            for b in range(_NBUF):
                scat[b].wait()

                @pl.when(g + 1 < n_gr)
                def _():
                    idx_copy(c0 + _NBUF + b, b).start()
                    val_copy(c0 + _NBUF + b, b).start()
            return carry

        lax.fori_loop(0, n_gr, group, 0)
        plsc.subcore_barrier()

        @pl.when(sid < _NS - 1)
        def _():
            pltpu.sync_copy(acc_sh.at[pl.ds(sid * rpt, rpt)],
                            out_hbm.at[cid, pl.ds(sid * rpt, rpt)])

        @pl.when(sid == _NS - 1)
        def _():
            pltpu.sync_copy(acc_sh.at[pl.ds((_NS - 1) * rpt, tail)],
                            out_hbm.at[cid, pl.ds((_NS - 1) * rpt, tail)])

    return sk(vals, idx, zeros_nd)


_BS = 1600


def _tc_matmul(a, w):
    m, k = a.shape
    n = w.shape[1]
    bs = 2000

    def body(a_ref, w_ref, o_ref):
        o_ref[...] = jnp.dot(a_ref[...], w_ref[...],
                             preferred_element_type=jnp.float32)

    return pl.pallas_call(
        body,
        grid=(m // bs,),
        in_specs=[_row_block(bs, k), _rep_block((k, n))],
        out_specs=_row_block(bs, n),
        out_shape=jax.ShapeDtypeStruct((m, n), jnp.float32),
    )(a, w)


def _tc_init_h(g, ea, wie, bi, wh, off, e_full, pb_prev):
    eh, h = g.shape
    de = ea.shape[1]
    blk_off = off // _BS
    grid = eh // _BS

    def body(g_ref, ea_ref, wie_ref, bi_ref, wh_ref, *rest):
        if pb_prev is not None:
            rest = rest[1:]
        h0_ref, p_ref, pb_ref = rest
        h0 = (g_ref[...]
              + jnp.dot(ea_ref[...], wie_ref[...],
                        preferred_element_type=jnp.float32)
              + bi_ref[...])
        h0_ref[...] = h0
        p = jnp.dot(jnp.maximum(h0, 0.0), wh_ref[...],
                    preferred_element_type=jnp.float32)
        p_ref[...] = p
        pb_ref[...] = p

    in_specs = [_row_block(_BS, h), _off_block(_BS, de, blk_off),
                _rep_block((de, h)), _rep_block((1, h)), _rep_block((h, h))]
    args = [g, ea, wie, bi, wh]
    aliases = {}
    if pb_prev is not None:
        in_specs.append(_rep_block((8, h)))
        args.append(pb_prev)
        aliases = {5: 2}
    return pl.pallas_call(
        body,
        grid=(grid,),
        in_specs=in_specs,
        out_specs=[_row_block(_BS, h), _row_block(_BS, h),
                   _off_block(_BS, h, blk_off)],
        out_shape=[jax.ShapeDtypeStruct((eh, h), jnp.float32),
                   jax.ShapeDtypeStruct((eh, h), jnp.float32),
                   jax.ShapeDtypeStruct((e_full, h), jnp.float32)],
        input_output_aliases=aliases,
    )(*args)


def _tc_step_h(h0, a, b, bhv, wh, off, e_full, pb_prev):
    eh, h = h0.shape
    blk_off = off // _BS
    grid = eh // _BS

    def body(h0_ref, a_ref, b_ref, bh_ref, wh_ref, *rest):
        if pb_prev is not None:
            rest = rest[1:]
        p_ref, pb_ref = rest
        ht = jnp.maximum(h0_ref[...] + a_ref[...] - b_ref[...] + bh_ref[...],
                         0.0)
        p = jnp.dot(ht, wh_ref[...], preferred_element_type=jnp.float32)
        p_ref[...] = p
        pb_ref[...] = p

    in_specs = [_row_block(_BS, h)] * 3 + [_rep_block((1, h)),
                                           _rep_block((h, h))]
    args = [h0, a, b, bhv, wh]
    aliases = {}
    if pb_prev is not None:
        in_specs.append(_rep_block((8, h)))
        args.append(pb_prev)
        aliases = {5: 1}
    return pl.pallas_call(
        body,
        grid=(grid,),
        in_specs=in_specs,
        out_specs=[_row_block(_BS, h), _off_block(_BS, h, blk_off)],
        out_shape=[jax.ShapeDtypeStruct((eh, h), jnp.float32),
                   jax.ShapeDtypeStruct((e_full, h), jnp.float32)],
        input_output_aliases=aliases,
    )(*args)


def _tc_last_h(h0, a, b, bhv):
    eh, h = h0.shape

    def body(h0_ref, a_ref, b_ref, bh_ref, o_ref):
        o_ref[...] = jnp.maximum(
            h0_ref[...] + a_ref[...] - b_ref[...] + bh_ref[...], 0.0)

    return pl.pallas_call(
        body,
        grid=(eh // _BS,),
        in_specs=[_row_block(_BS, h)] * 3 + [_rep_block((1, h))],
        out_specs=_row_block(_BS, h),
        out_shape=jax.ShapeDtypeStruct((eh, h), jnp.float32),
    )(h0, a, b, bhv)


def _tc_merge4(q0, q1, q2, q3):
    n, h = q0.shape
    bs = 2000

    def body(a_ref, b_ref, c_ref, d_ref, o_ref):
        o_ref[...] = ((a_ref[...] + b_ref[...])
                      + (c_ref[...] + d_ref[...]))

    return pl.pallas_call(
        body,
        grid=(n // bs,),
        in_specs=[_row_block(bs, h)] * 4,
        out_specs=_row_block(bs, h),
        out_shape=jax.ShapeDtypeStruct((n, h), jnp.float32),
    )(q0, q1, q2, q3)


def _tc_final(q0, q1, q2, q3, x, wox, wom, bo):
    n, h = x.shape
    bs = 2000

    def body(q0_ref, q1_ref, q2_ref, q3_ref, x_ref, wox_ref, wom_ref,
             bo_ref, o_ref):
        ns = (q0_ref[...] + q1_ref[...]) + (q2_ref[...] + q3_ref[...])
        s = jnp.sum(ns, axis=1, keepdims=True)
        m = jnp.where(s == 0.0, x_ref[...], ns)
        o_ref[...] = jnp.maximum(
            jnp.dot(x_ref[...], wox_ref[...],
                    preferred_element_type=jnp.float32)
            + jnp.dot(m, wom_ref[...], preferred_element_type=jnp.float32)
            + bo_ref[...], 0.0)

    return pl.pallas_call(
        body,
        grid=(n // bs,),
        in_specs=[_row_block(bs, h)] * 5 + [_rep_block((h, h))] * 2
                 + [_rep_block((1, h))],
        out_specs=_row_block(bs, h),
        out_shape=jax.ShapeDtypeStruct((n, h), jnp.float32),
    )(q0, q1, q2, q3, x, wox, wom, bo)


def kernel(x, edge_index, rev_edge_index, edge_attr, W_i, b_i, W_h, b_h,
           W_o, b_o):
    n, df = x.shape
    h = W_h.shape[0]
    e = edge_attr.shape[0]
    e_lo = 153600
    e_hi = e - e_lo
    src = edge_index[0]
    dst = edge_index[1]
    rev = rev_edge_index
    wi_x, wi_e = W_i[:df], W_i[df:]
    wo_x, wo_m = W_o[:df], W_o[df:]
    bi = b_i.reshape(1, h)
    bhv = b_h.reshape(1, h)
    bo = b_o.reshape(1, h)
    zeros_nd = jnp.zeros((n, h), jnp.float32)

    hx = _tc_matmul(x, wi_x)
    g0l = _sc_gather1h(hx, src, 0, e_lo)
    g0h = _sc_gather1h(hx, src, e_lo, e_hi)
    h0l, p_l, pb = _tc_init_h(g0l, edge_attr, wi_e, bi, W_h, 0, e, None)
    h0h, p_h, pb = _tc_init_h(g0h, edge_attr, wi_e, bi, W_h, e_lo, e, pb)

    for _ in range(_DEPTH - 2):
        parts1 = _sc_scatter_h(p_l, dst, 0, zeros_nd)
        parts2 = _sc_scatter_h(p_h, dst, e_lo, zeros_nd)
        ns = _tc_merge4(parts1[0], parts1[1], parts2[0], parts2[1])
        a_l, b_l = _sc_gather2h(ns, src, pb, rev, 0, e_lo)
        p_l, pb2 = _tc_step_h(h0l, a_l, b_l, bhv, W_h, 0, e, None)
        a_h, b_h2 = _sc_gather2h(ns, src, pb, rev, e_lo, e_hi)
        p_h, pb = _tc_step_h(h0h, a_h, b_h2, bhv, W_h, e_lo, e, pb2)

    parts1 = _sc_scatter_h(p_l, dst, 0, zeros_nd)
    parts2 = _sc_scatter_h(p_h, dst, e_lo, zeros_nd)
    ns = _tc_merge4(parts1[0], parts1[1], parts2[0], parts2[1])
    a_l, b_l = _sc_gather2h(ns, src, pb, rev, 0, e_lo)
    hf_l = _tc_last_h(h0l, a_l, b_l, bhv)
    a_h, b_h2 = _sc_gather2h(ns, src, pb, rev, e_lo, e_hi)
    hf_h = _tc_last_h(h0h, a_h, b_h2, bhv)

    parts1 = _sc_scatter_h(hf_l, dst, 0, zeros_nd)
    parts2 = _sc_scatter_h(hf_h, dst, e_lo, zeros_nd)
    return _tc_final(parts1[0], parts1[1], parts2[0], parts2[1],
                     x, wo_x, wo_m, bo)


# speedup vs baseline: 1.0742x; 1.0742x over previous
import functools

import jax
import jax.numpy as jnp
from jax import lax
from jax.experimental import pallas as pl
from jax.experimental.pallas import tpu as pltpu
from jax.experimental.pallas import tpu_sc as plsc

_DEPTH = 3
_NC = 2
_NS = 16
_NW = _NC * _NS
_CW = 80
_SW = 40
_NBUF = 5


def _row_block(bs, d):
    return pl.BlockSpec((bs, d), lambda i: (i, 0))


def _off_block(bs, d, blk_off):
    return pl.BlockSpec((bs, d), lambda i, o=blk_off: (o + i, 0))


def _rep_block(shape):
    return pl.BlockSpec(shape, lambda i: tuple(0 for _ in shape))


def _sc_gather1h(table, idx, off, eh):
    d = table.shape[1]
    per_w = eh // _NW
    n_ch = per_w // _CW
    n_gr = n_ch // _NBUF
    mesh = plsc.VectorSubcoreMesh(core_axis_name="c", subcore_axis_name="s")

    @functools.partial(
        pl.kernel,
        out_type=jax.ShapeDtypeStruct((eh, d), jnp.float32),
        mesh=mesh,
        scratch_types=[
            pltpu.VMEM((_NBUF, _CW), jnp.int32),
            pltpu.VMEM((_NBUF, _CW, d), jnp.float32),
            pltpu.SemaphoreType.DMA((_NBUF,)),
            pltpu.SemaphoreType.DMA((_NBUF,)),
            pltpu.SemaphoreType.DMA((_NBUF,)),
        ],
    )
    def gk(table_hbm, idx_hbm, out_hbm, idx_v, rows_v, isem, gsem, ssem):
        wid = lax.axis_index("s") * _NC + lax.axis_index("c")
        base = wid * per_w

        def idx_copy(c, b):
            return pltpu.make_async_copy(
                idx_hbm.at[pl.ds(off + base + c * _CW, _CW)], idx_v.at[b],
                isem.at[b])

        def out_copy(c, b):
            return pltpu.make_async_copy(
                rows_v.at[b], out_hbm.at[pl.ds(base + c * _CW, _CW)],
                ssem.at[b])

        for b in range(_NBUF):
            idx_copy(b, b).start()

        def group(g, carry):
            c0 = g * _NBUF
            gathers = []
            for b in range(_NBUF):
                @pl.when(g > 0)
                def _():
                    out_copy(0, b).wait()

                idx_copy(c0 + b, b).wait()
                gathers.append(pltpu.async_copy(
                    table_hbm.at[idx_v.at[b]], rows_v.at[b], gsem.at[b]))
            for b in range(_NBUF):
                gathers[b].wait()
                out_copy(c0 + b, b).start()

                @pl.when(g + 1 < n_gr)
                def _():
                    idx_copy(c0 + _NBUF + b, b).start()
            return carry

        lax.fori_loop(0, n_gr, group, 0)
        for b in range(_NBUF):
            out_copy(0, b).wait()

    return gk(table, idx)


def _sc_gather2h(table_a, idx_a, table_b, idx_b, off, eh):
    d = table_a.shape[1]
    per_w = eh // _NW
    n_ch = per_w // _CW
    n_gr = n_ch // _NBUF
    mesh = plsc.VectorSubcoreMesh(core_axis_name="c", subcore_axis_name="s")

    @functools.partial(
        pl.kernel,
        out_type=(jax.ShapeDtypeStruct((eh, d), jnp.float32),
                  jax.ShapeDtypeStruct((eh, d), jnp.float32)),
        mesh=mesh,
        scratch_types=[
            pltpu.VMEM((_NBUF, _CW), jnp.int32),
            pltpu.VMEM((_NBUF, _CW), jnp.int32),
            pltpu.VMEM((_NBUF, _CW, d), jnp.float32),
            pltpu.VMEM((_NBUF, _CW, d), jnp.float32),
            pltpu.SemaphoreType.DMA((_NBUF,)),
            pltpu.SemaphoreType.DMA((_NBUF,)),
            pltpu.SemaphoreType.DMA((_NBUF,)),
            pltpu.SemaphoreType.DMA((_NBUF,)),
            pltpu.SemaphoreType.DMA((_NBUF,)),
            pltpu.SemaphoreType.DMA((_NBUF,)),
        ],
    )
    def gk(ta_hbm, ia_hbm, tb_hbm, ib_hbm, outa_hbm, outb_hbm,
           ia_v, ib_v, ra_v, rb_v, isema, isemb, gsema, gsemb, ssema, ssemb):
        wid = lax.axis_index("s") * _NC + lax.axis_index("c")
        base = wid * per_w

        def icopy(hbm, vref, sem, c, b):
            return pltpu.make_async_copy(
                hbm.at[pl.ds(off + base + c * _CW, _CW)], vref.at[b],
                sem.at[b])

        def ocopy(vref, hbm, sem, c, b):
            return pltpu.make_async_copy(
                vref.at[b], hbm.at[pl.ds(base + c * _CW, _CW)], sem.at[b])

        for b in range(_NBUF):
            icopy(ia_hbm, ia_v, isema, b, b).start()
            icopy(ib_hbm, ib_v, isemb, b, b).start()

        def group(g, carry):
            c0 = g * _NBUF
            gathers = []
            for b in range(_NBUF):
                @pl.when(g > 0)
                def _():
                    ocopy(ra_v, outa_hbm, ssema, 0, b).wait()
                    ocopy(rb_v, outb_hbm, ssemb, 0, b).wait()

                icopy(ia_hbm, ia_v, isema, c0 + b, b).wait()
                icopy(ib_hbm, ib_v, isemb, c0 + b, b).wait()
                gathers.append((
                    pltpu.async_copy(ta_hbm.at[ia_v.at[b]], ra_v.at[b],
                                     gsema.at[b]),
                    pltpu.async_copy(tb_hbm.at[ib_v.at[b]], rb_v.at[b],
                                     gsemb.at[b])))
            for b in range(_NBUF):
                ga, gb = gathers[b]
                ga.wait()
                ocopy(ra_v, outa_hbm, ssema, c0 + b, b).start()
                gb.wait()
                ocopy(rb_v, outb_hbm, ssemb, c0 + b, b).start()

                @pl.when(g + 1 < n_gr)
                def _():
                    icopy(ia_hbm, ia_v, isema, c0 + _NBUF + b, b).start()
                    icopy(ib_hbm, ib_v, isemb, c0 + _NBUF + b, b).start()
            return carry

        lax.fori_loop(0, n_gr, group, 0)
        for b in range(_NBUF):
            ocopy(ra_v, outa_hbm, ssema, 0, b).wait()
            ocopy(rb_v, outb_hbm, ssemb, 0, b).wait()

    return gk(table_a, idx_a, table_b, idx_b)


def _sc_scatter_h(vals, idx, off, zeros_nd):
    eh, d = vals.shape
    n = zeros_nd.shape[0]
    per_w = eh // _NW
    n_ch = per_w // _SW
    n_gr = n_ch // _NBUF
    rpt = 640
    tail = n - (_NS - 1) * rpt
    mesh = plsc.VectorSubcoreMesh(core_axis_name="c", subcore_axis_name="s")

    @functools.partial(
        pl.kernel,
        out_type=jax.ShapeDtypeStruct((_NC, n, d), jnp.float32),
        mesh=mesh,
        scratch_types=[
            pltpu.VMEM((_NBUF, _SW), jnp.int32),
            pltpu.VMEM((_NBUF, _SW, d), jnp.float32),
            pltpu.VMEM_SHARED((n, d), jnp.float32),
            pltpu.SemaphoreType.DMA((_NBUF,)),
            pltpu.SemaphoreType.DMA((_NBUF,)),
            pltpu.SemaphoreType.DMA((_NBUF,)),
        ],
    )
    def sk(vals_hbm, idx_hbm, zeros_hbm, out_hbm, idx_v, rows_v, acc_sh,
           isem, vsem, ssem):
        cid = lax.axis_index("c")
        sid = lax.axis_index("s")
        wid = sid * _NC + cid
        base = wid * per_w

        @pl.when(sid < _NS - 1)
        def _():
            pltpu.sync_copy(zeros_hbm.at[pl.ds(sid * rpt, rpt)],
                            acc_sh.at[pl.ds(sid * rpt, rpt)])

        @pl.when(sid == _NS - 1)
        def _():
            pltpu.sync_copy(zeros_hbm.at[pl.ds((_NS - 1) * rpt, tail)],
                            acc_sh.at[pl.ds((_NS - 1) * rpt, tail)])

        plsc.subcore_barrier()

        def idx_copy(c, b):
            return pltpu.make_async_copy(
                idx_hbm.at[pl.ds(off + base + c * _SW, _SW)], idx_v.at[b],
                isem.at[b])

        def val_copy(c, b):
            return pltpu.make_async_copy(
                vals_hbm.at[pl.ds(base + c * _SW, _SW)], rows_v.at[b],
                vsem.at[b])

        for b in range(_NBUF):
            idx_copy(b, b).start()
            val_copy(b, b).start()

        def group(g, carry):
            c0 = g * _NBUF
            scat = []
            for b in range(_NBUF):
                idx_copy(c0 + b, b).wait()
                val_copy(c0 + b, b).wait()
                scat.append(pltpu.async_copy(
                    rows_v.at[b], acc_sh.at[idx_v.at[b]], ssem.at[b],
                    add=True))
            for b in range(_NBUF):
                scat[b].wait()

                @pl.when(g + 1 < n_gr)
                def _():
                    idx_copy(c0 + _NBUF + b, b).start()
                    val_copy(c0 + _NBUF + b, b).start()
            return carry

        lax.fori_loop(0, n_gr, group, 0)
        plsc.subcore_barrier()

        @pl.when(sid < _NS - 1)
        def _():
            pltpu.sync_copy(acc_sh.at[pl.ds(sid * rpt, rpt)],
                            out_hbm.at[cid, pl.ds(sid * rpt, rpt)])

        @pl.when(sid == _NS - 1)
        def _():
            pltpu.sync_copy(acc_sh.at[pl.ds((_NS - 1) * rpt, tail)],
                            out_hbm.at[cid, pl.ds((_NS - 1) * rpt, tail)])

    return sk(vals, idx, zeros_nd)


def _tc_matmul(a, w):
    m, k = a.shape
    n = w.shape[1]
    bs = 2000

    def body(a_ref, w_ref, o_ref):
        o_ref[...] = jnp.dot(a_ref[...], w_ref[...],
                             preferred_element_type=jnp.float32)

    return pl.pallas_call(
        body,
        grid=(m // bs,),
        in_specs=[_row_block(bs, k), _rep_block((k, n))],
        out_specs=_row_block(bs, n),
        out_shape=jax.ShapeDtypeStruct((m, n), jnp.float32),
    )(a, w)


def _tc_init(g, ea, wie, bi, wh):
    e, h = g.shape
    de = ea.shape[1]
    bs = 2560

    def body(g_ref, ea_ref, wie_ref, bi_ref, wh_ref, h0_ref, p_ref):
        h0 = (g_ref[...]
              + jnp.dot(ea_ref[...], wie_ref[...],
                        preferred_element_type=jnp.float32)
              + bi_ref[...])
        h0_ref[...] = h0
        p_ref[...] = jnp.dot(jnp.maximum(h0, 0.0), wh_ref[...],
                             preferred_element_type=jnp.float32)

    return pl.pallas_call(
        body,
        grid=(e // bs,),
        in_specs=[_row_block(bs, h), _row_block(bs, de), _rep_block((de, h)),
                  _rep_block((1, h)), _rep_block((h, h))],
        out_specs=[_row_block(bs, h), _row_block(bs, h)],
        out_shape=[jax.ShapeDtypeStruct((e, h), jnp.float32),
                   jax.ShapeDtypeStruct((e, h), jnp.float32)],
    )(g, ea, wie, bi, wh)


def _tc_step(h0, a, b, bhv, wh):
    e, h = h0.shape
    bs = 2560

    def body(h0_ref, a_ref, b_ref, bh_ref, wh_ref, p_ref):
        ht = jnp.maximum(h0_ref[...] + a_ref[...] - b_ref[...] + bh_ref[...],
                         0.0)
        p_ref[...] = jnp.dot(ht, wh_ref[...],
                             preferred_element_type=jnp.float32)

    return pl.pallas_call(
        body,
        grid=(e // bs,),
        in_specs=[_row_block(bs, h)] * 3 + [_rep_block((1, h)),
                                            _rep_block((h, h))],
        out_specs=_row_block(bs, h),
        out_shape=jax.ShapeDtypeStruct((e, h), jnp.float32),
    )(h0, a, b, bhv, wh)


def _tc_last(h0, a, b, bhv):
    e, h = h0.shape
    bs = 2560

    def body(h0_ref, a_ref, b_ref, bh_ref, o_ref):
        o_ref[...] = jnp.maximum(
            h0_ref[...] + a_ref[...] - b_ref[...] + bh_ref[...], 0.0)

    return pl.pallas_call(
        body,
        grid=(e // bs,),
        in_specs=[_row_block(bs, h)] * 3 + [_rep_block((1, h))],
        out_specs=_row_block(bs, h),
        out_shape=jax.ShapeDtypeStruct((e, h), jnp.float32),
    )(h0, a, b, bhv)


def _tc_merge(p0, p1):
    n, h = p0.shape
    bs = 2000

    def body(a_ref, b_ref, o_ref):
        o_ref[...] = a_ref[...] + b_ref[...]

    return pl.pallas_call(
        body,
        grid=(n // bs,),
        in_specs=[_row_block(bs, h)] * 2,
        out_specs=_row_block(bs, h),
        out_shape=jax.ShapeDtypeStruct((n, h), jnp.float32),
    )(p0, p1)


def _tc_final(p0, p1, x, wox, wom, bo):
    n, h = x.shape
    bs = 2000

    def body(p0_ref, p1_ref, x_ref, wox_ref, wom_ref, bo_ref, o_ref):
        ns = p0_ref[...] + p1_ref[...]
        s = jnp.sum(ns, axis=1, keepdims=True)
        m = jnp.where(s == 0.0, x_ref[...], ns)
        o_ref[...] = jnp.maximum(
            jnp.dot(x_ref[...], wox_ref[...],
                    preferred_element_type=jnp.float32)
            + jnp.dot(m, wom_ref[...], preferred_element_type=jnp.float32)
            + bo_ref[...], 0.0)

    return pl.pallas_call(
        body,
        grid=(n // bs,),
        in_specs=[_row_block(bs, h)] * 3 + [_rep_block((h, h))] * 2
                 + [_rep_block((1, h))],
        out_specs=_row_block(bs, h),
        out_shape=jax.ShapeDtypeStruct((n, h), jnp.float32),
    )(p0, p1, x, wox, wom, bo)


def kernel(x, edge_index, rev_edge_index, edge_attr, W_i, b_i, W_h, b_h,
           W_o, b_o):
    n, df = x.shape
    h = W_h.shape[0]
    e = edge_attr.shape[0]
    src = edge_index[0]
    dst = edge_index[1]
    rev = rev_edge_index
    wi_x, wi_e = W_i[:df], W_i[df:]
    wo_x, wo_m = W_o[:df], W_o[df:]
    bi = b_i.reshape(1, h)
    bhv = b_h.reshape(1, h)
    bo = b_o.reshape(1, h)
    zeros_nd = jnp.zeros((n, h), jnp.float32)

    hx = _tc_matmul(x, wi_x)
    g0 = _sc_gather1h(hx, src, 0, e)
    h0, p = _tc_init(g0, edge_attr, wi_e, bi, W_h)

    for _ in range(_DEPTH - 2):
        parts = _sc_scatter_h(p, dst, 0, zeros_nd)
        ns = _tc_merge(parts[0], parts[1])
        a, b = _sc_gather2h(ns, src, p, rev, 0, e)
        p = _tc_step(h0, a, b, bhv, W_h)

    parts = _sc_scatter_h(p, dst, 0, zeros_nd)
    ns = _tc_merge(parts[0], parts[1])
    a, b = _sc_gather2h(ns, src, p, rev, 0, e)
    h_fin = _tc_last(h0, a, b, bhv)

    parts = _sc_scatter_h(h_fin, dst, 0, zeros_nd)
    return _tc_final(parts[0], parts[1], x, wo_x, wo_m, bo)


# speedup vs baseline: 1.1117x; 1.0349x over previous
import functools

import jax
import jax.numpy as jnp
from jax import lax
from jax.experimental import pallas as pl
from jax.experimental.pallas import tpu as pltpu
from jax.experimental.pallas import tpu_sc as plsc

_DEPTH = 3
_NC = 2
_NS = 16
_NW = _NC * _NS
_CW = 80
_SW = 40
_NBUF = 5


def _row_block(bs, d):
    return pl.BlockSpec((bs, d), lambda i: (i, 0))


def _off_block(bs, d, blk_off):
    return pl.BlockSpec((bs, d), lambda i, o=blk_off: (o + i, 0))


def _rep_block(shape):
    return pl.BlockSpec(shape, lambda i: tuple(0 for _ in shape))


def _sc_gather1h(table, idx, off, eh):
    d = table.shape[1]
    per_w = eh // _NW
    n_ch = per_w // _CW
    n_gr = n_ch // _NBUF
    mesh = plsc.VectorSubcoreMesh(core_axis_name="c", subcore_axis_name="s")

    @functools.partial(
        pl.kernel,
        out_type=jax.ShapeDtypeStruct((eh, d), jnp.float32),
        mesh=mesh,
        scratch_types=[
            pltpu.VMEM((_NBUF, _CW), jnp.int32),
            pltpu.VMEM((_NBUF, _CW, d), jnp.float32),
            pltpu.SemaphoreType.DMA((_NBUF,)),
            pltpu.SemaphoreType.DMA((_NBUF,)),
            pltpu.SemaphoreType.DMA((_NBUF,)),
        ],
    )
    def gk(table_hbm, idx_hbm, out_hbm, idx_v, rows_v, isem, gsem, ssem):
        wid = lax.axis_index("s") * _NC + lax.axis_index("c")
        base = wid * per_w

        def idx_copy(c, b):
            return pltpu.make_async_copy(
                idx_hbm.at[pl.ds(off + base + c * _CW, _CW)], idx_v.at[b],
                isem.at[b])

        def out_copy(c, b):
            return pltpu.make_async_copy(
                rows_v.at[b], out_hbm.at[pl.ds(base + c * _CW, _CW)],
                ssem.at[b])

        for b in range(_NBUF):
            idx_copy(b, b).start()

        def group(g, carry):
            c0 = g * _NBUF
            gathers = []
            for b in range(_NBUF):
                @pl.when(g > 0)
                def _():
                    out_copy(0, b).wait()

                idx_copy(c0 + b, b).wait()
                gathers.append(pltpu.async_copy(
                    table_hbm.at[idx_v.at[b]], rows_v.at[b], gsem.at[b]))
            for b in range(_NBUF):
                gathers[b].wait()
                out_copy(c0 + b, b).start()

                @pl.when(g + 1 < n_gr)
                def _():
                    idx_copy(c0 + _NBUF + b, b).start()
            return carry

        lax.fori_loop(0, n_gr, group, 0)
        for b in range(_NBUF):
            out_copy(0, b).wait()

    return gk(table, idx)


def _sc_gather2h(table_a, idx_a, table_b, idx_b, off, eh):
    d = table_a.shape[1]
    per_w = eh // _NW
    n_ch = per_w // _CW
    n_gr = n_ch // _NBUF
    mesh = plsc.VectorSubcoreMesh(core_axis_name="c", subcore_axis_name="s")

    @functools.partial(
        pl.kernel,
        out_type=(jax.ShapeDtypeStruct((eh, d), jnp.float32),
                  jax.ShapeDtypeStruct((eh, d), jnp.float32)),
        mesh=mesh,
        scratch_types=[
            pltpu.VMEM((_NBUF, _CW), jnp.int32),
            pltpu.VMEM((_NBUF, _CW), jnp.int32),
            pltpu.VMEM((_NBUF, _CW, d), jnp.float32),
            pltpu.VMEM((_NBUF, _CW, d), jnp.float32),
            pltpu.SemaphoreType.DMA((_NBUF,)),
            pltpu.SemaphoreType.DMA((_NBUF,)),
            pltpu.SemaphoreType.DMA((_NBUF,)),
            pltpu.SemaphoreType.DMA((_NBUF,)),
            pltpu.SemaphoreType.DMA((_NBUF,)),
            pltpu.SemaphoreType.DMA((_NBUF,)),
        ],
    )
    def gk(ta_hbm, ia_hbm, tb_hbm, ib_hbm, outa_hbm, outb_hbm,
           ia_v, ib_v, ra_v, rb_v, isema, isemb, gsema, gsemb, ssema, ssemb):
        wid = lax.axis_index("s") * _NC + lax.axis_index("c")
        base = wid * per_w

        def icopy(hbm, vref, sem, c, b):
            return pltpu.make_async_copy(
                hbm.at[pl.ds(off + base + c * _CW, _CW)], vref.at[b],
                sem.at[b])

        def ocopy(vref, hbm, sem, c, b):
            return pltpu.make_async_copy(
                vref.at[b], hbm.at[pl.ds(base + c * _CW, _CW)], sem.at[b])

        for b in range(_NBUF):
            icopy(ia_hbm, ia_v, isema, b, b).start()
            icopy(ib_hbm, ib_v, isemb, b, b).start()

        def group(g, carry):
            c0 = g * _NBUF
            gathers = []
            for b in range(_NBUF):
                @pl.when(g > 0)
                def _():
                    ocopy(ra_v, outa_hbm, ssema, 0, b).wait()
                    ocopy(rb_v, outb_hbm, ssemb, 0, b).wait()

                icopy(ia_hbm, ia_v, isema, c0 + b, b).wait()
                icopy(ib_hbm, ib_v, isemb, c0 + b, b).wait()
                gathers.append((
                    pltpu.async_copy(ta_hbm.at[ia_v.at[b]], ra_v.at[b],
                                     gsema.at[b]),
                    pltpu.async_copy(tb_hbm.at[ib_v.at[b]], rb_v.at[b],
                                     gsemb.at[b])))
            for b in range(_NBUF):
                ga, gb = gathers[b]
                ga.wait()
                ocopy(ra_v, outa_hbm, ssema, c0 + b, b).start()
                gb.wait()
                ocopy(rb_v, outb_hbm, ssemb, c0 + b, b).start()

                @pl.when(g + 1 < n_gr)
                def _():
                    icopy(ia_hbm, ia_v, isema, c0 + _NBUF + b, b).start()
                    icopy(ib_hbm, ib_v, isemb, c0 + _NBUF + b, b).start()
            return carry

        lax.fori_loop(0, n_gr, group, 0)
        for b in range(_NBUF):
            ocopy(ra_v, outa_hbm, ssema, 0, b).wait()
            ocopy(rb_v, outb_hbm, ssemb, 0, b).wait()

    return gk(table_a, idx_a, table_b, idx_b)


def _sc_scatter_h(vals, idx, off, zeros_nd):
    eh, d = vals.shape
    n = zeros_nd.shape[0]
    per_w = eh // _NW
    n_ch = per_w // _SW
    n_gr = n_ch // _NBUF
    rpt = 640
    tail = n - (_NS - 1) * rpt
    mesh = plsc.VectorSubcoreMesh(core_axis_name="c", subcore_axis_name="s")

    @functools.partial(
        pl.kernel,
        out_type=jax.ShapeDtypeStruct((_NC, n, d), jnp.float32),
        mesh=mesh,
        scratch_types=[
            pltpu.VMEM((_NBUF, _SW), jnp.int32),
            pltpu.VMEM((_NBUF, _SW, d), jnp.float32),
            pltpu.VMEM_SHARED((n, d), jnp.float32),
            pltpu.SemaphoreType.DMA((_NBUF,)),
            pltpu.SemaphoreType.DMA((_NBUF,)),
            pltpu.SemaphoreType.DMA((_NBUF,)),
        ],
    )
    def sk(vals_hbm, idx_hbm, zeros_hbm, out_hbm, idx_v, rows_v, acc_sh,
           isem, vsem, ssem):
        cid = lax.axis_index("c")
        sid = lax.axis_index("s")
        wid = sid * _NC + cid
        base = wid * per_w

        @pl.when(sid < _NS - 1)
        def _():
            pltpu.sync_copy(zeros_hbm.at[pl.ds(sid * rpt, rpt)],
                            acc_sh.at[pl.ds(sid * rpt, rpt)])

        @pl.when(sid == _NS - 1)
        def _():
            pltpu.sync_copy(zeros_hbm.at[pl.ds((_NS - 1) * rpt, tail)],
                            acc_sh.at[pl.ds((_NS - 1) * rpt, tail)])

        plsc.subcore_barrier()

        def idx_copy(c, b):
            return pltpu.make_async_copy(
                idx_hbm.at[pl.ds(off + base + c * _SW, _SW)], idx_v.at[b],
                isem.at[b])

        def val_copy(c, b):
            return pltpu.make_async_copy(
                vals_hbm.at[pl.ds(base + c * _SW, _SW)], rows_v.at[b],
                vsem.at[b])

        for b in range(_NBUF):
            idx_copy(b, b).start()
            val_copy(b, b).start()

        def group(g, carry):
            c0 = g * _NBUF
            scat = []
            for b in range(_NBUF):
                idx_copy(c0 + b, b).wait()
                val_copy(c0 + b, b).wait()
                scat.append(pltpu.async_copy(
                    rows_v.at[b], acc_sh.at[idx_v.at[b]], ssem.at[b],
                    add=True))
            for b in range(_NBUF):
                scat[b].wait()

                @pl.when(g + 1 < n_gr)
                def _():
                    idx_copy(c0 + _NBUF + b, b).start()
                    val_copy(c0 + _NBUF + b, b).start()
            return carry

        lax.fori_loop(0, n_gr, group, 0)
        plsc.subcore_barrier()

        @pl.when(sid < _NS - 1)
        def _():
            pltpu.sync_copy(acc_sh.at[pl.ds(sid * rpt, rpt)],
                            out_hbm.at[cid, pl.ds(sid * rpt, rpt)])

        @pl.when(sid == _NS - 1)
        def _():
            pltpu.sync_copy(acc_sh.at[pl.ds((_NS - 1) * rpt, tail)],
                            out_hbm.at[cid, pl.ds((_NS - 1) * rpt, tail)])

    return sk(vals, idx, zeros_nd)


def _tc_matmul(a, w):
    m, k = a.shape
    n = w.shape[1]
    bs = 2000

    def body(a_ref, w_ref, o_ref):
        o_ref[...] = jnp.dot(a_ref[...], w_ref[...],
                             preferred_element_type=jnp.float32)

    return pl.pallas_call(
        body,
        grid=(m // bs,),
        in_specs=[_row_block(bs, k), _rep_block((k, n))],
        out_specs=_row_block(bs, n),
        out_shape=jax.ShapeDtypeStruct((m, n), jnp.float32),
    )(a, w)


def _tc_init(g, ea, wie, bi, wh):
    e, h = g.shape
    de = ea.shape[1]
    bs = 2560

    def body(g_ref, ea_ref, wie_ref, bi_ref, wh_ref, h0_ref, p_ref):
        h0 = (g_ref[...]
              + jnp.dot(ea_ref[...], wie_ref[...],
                        preferred_element_type=jnp.float32)
              + bi_ref[...])
        h0_ref[...] = h0.astype(jnp.bfloat16)
        p_ref[...] = jnp.dot(jnp.maximum(h0, 0.0), wh_ref[...],
                             preferred_element_type=jnp.float32)

    return pl.pallas_call(
        body,
        grid=(e // bs,),
        in_specs=[_row_block(bs, h), _row_block(bs, de), _rep_block((de, h)),
                  _rep_block((1, h)), _rep_block((h, h))],
        out_specs=[_row_block(bs, h), _row_block(bs, h)],
        out_shape=[jax.ShapeDtypeStruct((e, h), jnp.bfloat16),
                   jax.ShapeDtypeStruct((e, h), jnp.float32)],
    )(g, ea, wie, bi, wh)


def _tc_step(h0, a, b, bhv, wh):
    e, h = h0.shape
    bs = 2560

    def body(h0_ref, a_ref, b_ref, bh_ref, wh_ref, p_ref):
        ht = jnp.maximum(h0_ref[...].astype(jnp.float32)
                         + a_ref[...] - b_ref[...] + bh_ref[...], 0.0)
        p_ref[...] = jnp.dot(ht, wh_ref[...],
                             preferred_element_type=jnp.float32)

    return pl.pallas_call(
        body,
        grid=(e // bs,),
        in_specs=[_row_block(bs, h)] * 3 + [_rep_block((1, h)),
                                            _rep_block((h, h))],
        out_specs=_row_block(bs, h),
        out_shape=jax.ShapeDtypeStruct((e, h), jnp.float32),
    )(h0, a, b, bhv, wh)


def _tc_last(h0, a, b, bhv):
    e, h = h0.shape
    bs = 2560

    def body(h0_ref, a_ref, b_ref, bh_ref, o_ref):
        o_ref[...] = jnp.maximum(
            h0_ref[...].astype(jnp.float32)
            + a_ref[...] - b_ref[...] + bh_ref[...], 0.0)

    return pl.pallas_call(
        body,
        grid=(e // bs,),
        in_specs=[_row_block(bs, h)] * 3 + [_rep_block((1, h))],
        out_specs=_row_block(bs, h),
        out_shape=jax.ShapeDtypeStruct((e, h), jnp.float32),
    )(h0, a, b, bhv)


def _tc_merge(p0, p1):
    n, h = p0.shape
    bs = 2000

    def body(a_ref, b_ref, o_ref):
        o_ref[...] = a_ref[...] + b_ref[...]

    return pl.pallas_call(
        body,
        grid=(n // bs,),
        in_specs=[_row_block(bs, h)] * 2,
        out_specs=_row_block(bs, h),
        out_shape=jax.ShapeDtypeStruct((n, h), jnp.float32),
    )(p0, p1)


def _tc_final(p0, p1, x, wox, wom, bo):
    n, h = x.shape
    bs = 2000

    def body(p0_ref, p1_ref, x_ref, wox_ref, wom_ref, bo_ref, o_ref):
        ns = p0_ref[...] + p1_ref[...]
        s = jnp.sum(ns, axis=1, keepdims=True)
        m = jnp.where(s == 0.0, x_ref[...], ns)
        o_ref[...] = jnp.maximum(
            jnp.dot(x_ref[...], wox_ref[...],
                    preferred_element_type=jnp.float32)
            + jnp.dot(m, wom_ref[...], preferred_element_type=jnp.float32)
            + bo_ref[...], 0.0)

    return pl.pallas_call(
        body,
        grid=(n // bs,),
        in_specs=[_row_block(bs, h)] * 3 + [_rep_block((h, h))] * 2
                 + [_rep_block((1, h))],
        out_specs=_row_block(bs, h),
        out_shape=jax.ShapeDtypeStruct((n, h), jnp.float32),
    )(p0, p1, x, wox, wom, bo)


def kernel(x, edge_index, rev_edge_index, edge_attr, W_i, b_i, W_h, b_h,
           W_o, b_o):
    n, df = x.shape
    h = W_h.shape[0]
    e = edge_attr.shape[0]
    src = edge_index[0]
    dst = edge_index[1]
    rev = rev_edge_index
    wi_x, wi_e = W_i[:df], W_i[df:]
    wo_x, wo_m = W_o[:df], W_o[df:]
    bi = b_i.reshape(1, h)
    bhv = b_h.reshape(1, h)
    bo = b_o.reshape(1, h)
    zeros_nd = jnp.zeros((n, h), jnp.float32)

    hx = _tc_matmul(x, wi_x)
    g0 = _sc_gather1h(hx, src, 0, e)
    h0, p = _tc_init(g0, edge_attr, wi_e, bi, W_h)

    for _ in range(_DEPTH - 2):
        parts = _sc_scatter_h(p, dst, 0, zeros_nd)
        ns = _tc_merge(parts[0], parts[1])
        a, b = _sc_gather2h(ns, src, p, rev, 0, e)
        p = _tc_step(h0, a, b, bhv, W_h)

    parts = _sc_scatter_h(p, dst, 0, zeros_nd)
    ns = _tc_merge(parts[0], parts[1])
    a, b = _sc_gather2h(ns, src, p, rev, 0, e)
    h_fin = _tc_last(h0, a, b, bhv)

    parts = _sc_scatter_h(h_fin, dst, 0, zeros_nd)
    return _tc_final(parts[0], parts[1], x, wo_x, wo_m, bo)


# speedup vs baseline: 1.1132x; 1.0014x over previous
import functools

import jax
import jax.numpy as jnp
from jax import lax
from jax.experimental import pallas as pl
from jax.experimental.pallas import tpu as pltpu
from jax.experimental.pallas import tpu_sc as plsc

_DEPTH = 3
_NC = 2
_NS = 16
_NW = _NC * _NS
_CW = 80
_SW = 80
_NBUF = 5


def _row_block(bs, d):
    return pl.BlockSpec((bs, d), lambda i: (i, 0))


def _off_block(bs, d, blk_off):
    return pl.BlockSpec((bs, d), lambda i, o=blk_off: (o + i, 0))


def _rep_block(shape):
    return pl.BlockSpec(shape, lambda i: tuple(0 for _ in shape))


def _sc_gather1h(table, idx, off, eh):
    d = table.shape[1]
    per_w = eh // _NW
    n_ch = per_w // _CW
    n_gr = n_ch // _NBUF
    mesh = plsc.VectorSubcoreMesh(core_axis_name="c", subcore_axis_name="s")

    @functools.partial(
        pl.kernel,
        out_type=jax.ShapeDtypeStruct((eh, d), jnp.float32),
        mesh=mesh,
        scratch_types=[
            pltpu.VMEM((_NBUF, _CW), jnp.int32),
            pltpu.VMEM((_NBUF, _CW, d), jnp.float32),
            pltpu.SemaphoreType.DMA((_NBUF,)),
            pltpu.SemaphoreType.DMA((_NBUF,)),
            pltpu.SemaphoreType.DMA((_NBUF,)),
        ],
    )
    def gk(table_hbm, idx_hbm, out_hbm, idx_v, rows_v, isem, gsem, ssem):
        wid = lax.axis_index("s") * _NC + lax.axis_index("c")
        base = wid * per_w

        def idx_copy(c, b):
            return pltpu.make_async_copy(
                idx_hbm.at[pl.ds(off + base + c * _CW, _CW)], idx_v.at[b],
                isem.at[b])

        def out_copy(c, b):
            return pltpu.make_async_copy(
                rows_v.at[b], out_hbm.at[pl.ds(base + c * _CW, _CW)],
                ssem.at[b])

        for b in range(_NBUF):
            idx_copy(b, b).start()

        def group(g, carry):
            c0 = g * _NBUF
            gathers = []
            for b in range(_NBUF):
                @pl.when(g > 0)
                def _():
                    out_copy(0, b).wait()

                idx_copy(c0 + b, b).wait()
                gathers.append(pltpu.async_copy(
                    table_hbm.at[idx_v.at[b]], rows_v.at[b], gsem.at[b]))
            for b in range(_NBUF):
                gathers[b].wait()
                out_copy(c0 + b, b).start()

                @pl.when(g + 1 < n_gr)
                def _():
                    idx_copy(c0 + _NBUF + b, b).start()
            return carry

        lax.fori_loop(0, n_gr, group, 0)
        for b in range(_NBUF):
            out_copy(0, b).wait()

    return gk(table, idx)


def _sc_gather2h(table_a, idx_a, table_b, idx_b, off, eh):
    d = table_a.shape[1]
    per_w = eh // _NW
    n_ch = per_w // _CW
    n_gr = n_ch // _NBUF
    mesh = plsc.VectorSubcoreMesh(core_axis_name="c", subcore_axis_name="s")

    @functools.partial(
        pl.kernel,
        out_type=(jax.ShapeDtypeStruct((eh, d), jnp.float32),
                  jax.ShapeDtypeStruct((eh, d), jnp.float32)),
        mesh=mesh,
        scratch_types=[
            pltpu.VMEM((_NBUF, _CW), jnp.int32),
            pltpu.VMEM((_NBUF, _CW), jnp.int32),
            pltpu.VMEM((_NBUF, _CW, d), jnp.float32),
            pltpu.VMEM((_NBUF, _CW, d), jnp.float32),
            pltpu.SemaphoreType.DMA((_NBUF,)),
            pltpu.SemaphoreType.DMA((_NBUF,)),
            pltpu.SemaphoreType.DMA((_NBUF,)),
            pltpu.SemaphoreType.DMA((_NBUF,)),
            pltpu.SemaphoreType.DMA((_NBUF,)),
            pltpu.SemaphoreType.DMA((_NBUF,)),
        ],
    )
    def gk(ta_hbm, ia_hbm, tb_hbm, ib_hbm, outa_hbm, outb_hbm,
           ia_v, ib_v, ra_v, rb_v, isema, isemb, gsema, gsemb, ssema, ssemb):
        wid = lax.axis_index("s") * _NC + lax.axis_index("c")
        base = wid * per_w

        def icopy(hbm, vref, sem, c, b):
            return pltpu.make_async_copy(
                hbm.at[pl.ds(off + base + c * _CW, _CW)], vref.at[b],
                sem.at[b])

        def ocopy(vref, hbm, sem, c, b):
            return pltpu.make_async_copy(
                vref.at[b], hbm.at[pl.ds(base + c * _CW, _CW)], sem.at[b])

        for b in range(_NBUF):
            icopy(ia_hbm, ia_v, isema, b, b).start()
            icopy(ib_hbm, ib_v, isemb, b, b).start()

        def group(g, carry):
            c0 = g * _NBUF
            gathers = []
            for b in range(_NBUF):
                @pl.when(g > 0)
                def _():
                    ocopy(ra_v, outa_hbm, ssema, 0, b).wait()
                    ocopy(rb_v, outb_hbm, ssemb, 0, b).wait()

                icopy(ia_hbm, ia_v, isema, c0 + b, b).wait()
                icopy(ib_hbm, ib_v, isemb, c0 + b, b).wait()
                gathers.append((
                    pltpu.async_copy(ta_hbm.at[ia_v.at[b]], ra_v.at[b],
                                     gsema.at[b]),
                    pltpu.async_copy(tb_hbm.at[ib_v.at[b]], rb_v.at[b],
                                     gsemb.at[b])))
            for b in range(_NBUF):
                ga, gb = gathers[b]
                ga.wait()
                ocopy(ra_v, outa_hbm, ssema, c0 + b, b).start()
                gb.wait()
                ocopy(rb_v, outb_hbm, ssemb, c0 + b, b).start()

                @pl.when(g + 1 < n_gr)
                def _():
                    icopy(ia_hbm, ia_v, isema, c0 + _NBUF + b, b).start()
                    icopy(ib_hbm, ib_v, isemb, c0 + _NBUF + b, b).start()
            return carry

        lax.fori_loop(0, n_gr, group, 0)
        for b in range(_NBUF):
            ocopy(ra_v, outa_hbm, ssema, 0, b).wait()
            ocopy(rb_v, outb_hbm, ssemb, 0, b).wait()

    return gk(table_a, idx_a, table_b, idx_b)


_SNB = 4


def _sc_scatter_h(vals, idx, off, zeros_nd):
    eh, d = vals.shape
    n = zeros_nd.shape[0]
    per_w = eh // _NW
    n_ch = per_w // _SW
    n_gr = (n_ch - 1) // _SNB
    rpt = 640
    tail = n - (_NS - 1) * rpt
    mesh = plsc.VectorSubcoreMesh(core_axis_name="c", subcore_axis_name="s")

    @functools.partial(
        pl.kernel,
        out_type=jax.ShapeDtypeStruct((_NC, n, d), jnp.float32),
        mesh=mesh,
        scratch_types=[
            pltpu.VMEM((_SNB, _SW), jnp.int32),
            pltpu.VMEM((_SNB, _SW, d), jnp.float32),
            pltpu.VMEM_SHARED((n, d), jnp.float32),
            pltpu.SemaphoreType.DMA((_SNB,)),
            pltpu.SemaphoreType.DMA((_SNB,)),
            pltpu.SemaphoreType.DMA((_SNB,)),
        ],
    )
    def sk(vals_hbm, idx_hbm, zeros_hbm, out_hbm, idx_v, rows_v, acc_sh,
           isem, vsem, ssem):
        cid = lax.axis_index("c")
        sid = lax.axis_index("s")
        wid = sid * _NC + cid
        base = wid * per_w

        @pl.when(sid < _NS - 1)
        def _():
            pltpu.sync_copy(zeros_hbm.at[pl.ds(sid * rpt, rpt)],
                            acc_sh.at[pl.ds(sid * rpt, rpt)])

        @pl.when(sid == _NS - 1)
        def _():
            pltpu.sync_copy(zeros_hbm.at[pl.ds((_NS - 1) * rpt, tail)],
                            acc_sh.at[pl.ds((_NS - 1) * rpt, tail)])

        plsc.subcore_barrier()

        def idx_copy(c, b):
            return pltpu.make_async_copy(
                idx_hbm.at[pl.ds(off + base + c * _SW, _SW)], idx_v.at[b],
                isem.at[b])

        def val_copy(c, b):
            return pltpu.make_async_copy(
                vals_hbm.at[pl.ds(base + c * _SW, _SW)], rows_v.at[b],
                vsem.at[b])

        for b in range(_SNB):
            idx_copy(b, b).start()
            val_copy(b, b).start()

        def group(g, carry):
            c0 = g * _SNB
            scat = []
            for b in range(_SNB):
                idx_copy(c0 + b, b).wait()
                val_copy(c0 + b, b).wait()
                scat.append(pltpu.async_copy(
                    rows_v.at[b], acc_sh.at[idx_v.at[b]], ssem.at[b],
                    add=True))
            for b in range(_SNB):
                scat[b].wait()

                @pl.when(g + 1 < n_gr)
                def _():
                    idx_copy(c0 + _SNB + b, b).start()
                    val_copy(c0 + _SNB + b, b).start()
            return carry

        lax.fori_loop(0, n_gr, group, 0)
        idx_copy(n_ch - 1, 0).start()
        val_copy(n_ch - 1, 0).start()
        idx_copy(n_ch - 1, 0).wait()
        val_copy(n_ch - 1, 0).wait()
        pltpu.async_copy(rows_v.at[0], acc_sh.at[idx_v.at[0]], ssem.at[0],
                         add=True).wait()
        plsc.subcore_barrier()

        @pl.when(sid < _NS - 1)
        def _():
            pltpu.sync_copy(acc_sh.at[pl.ds(sid * rpt, rpt)],
                            out_hbm.at[cid, pl.ds(sid * rpt, rpt)])

        @pl.when(sid == _NS - 1)
        def _():
            pltpu.sync_copy(acc_sh.at[pl.ds((_NS - 1) * rpt, tail)],
                            out_hbm.at[cid, pl.ds((_NS - 1) * rpt, tail)])

    return sk(vals, idx, zeros_nd)


def _tc_matmul(a, w):
    m, k = a.shape
    n = w.shape[1]
    bs = 2000

    def body(a_ref, w_ref, o_ref):
        o_ref[...] = jnp.dot(a_ref[...], w_ref[...],
                             preferred_element_type=jnp.float32)

    return pl.pallas_call(
        body,
        grid=(m // bs,),
        in_specs=[_row_block(bs, k), _rep_block((k, n))],
        out_specs=_row_block(bs, n),
        out_shape=jax.ShapeDtypeStruct((m, n), jnp.float32),
    )(a, w)


def _tc_init(g, ea, wie, bi, wh):
    e, h = g.shape
    de = ea.shape[1]
    bs = 2560

    def body(g_ref, ea_ref, wie_ref, bi_ref, wh_ref, h0_ref, p_ref):
        h0 = (g_ref[...]
              + jnp.dot(ea_ref[...], wie_ref[...],
                        preferred_element_type=jnp.float32)
              + bi_ref[...])
        h0_ref[...] = h0.astype(jnp.bfloat16)
        p_ref[...] = jnp.dot(jnp.maximum(h0, 0.0), wh_ref[...],
                             preferred_element_type=jnp.float32)

    return pl.pallas_call(
        body,
        grid=(e // bs,),
        in_specs=[_row_block(bs, h), _row_block(bs, de), _rep_block((de, h)),
                  _rep_block((1, h)), _rep_block((h, h))],
        out_specs=[_row_block(bs, h), _row_block(bs, h)],
        out_shape=[jax.ShapeDtypeStruct((e, h), jnp.bfloat16),
                   jax.ShapeDtypeStruct((e, h), jnp.float32)],
    )(g, ea, wie, bi, wh)


def _tc_step(h0, a, b, bhv, wh):
    e, h = h0.shape
    bs = 2560

    def body(h0_ref, a_ref, b_ref, bh_ref, wh_ref, p_ref):
        ht = jnp.maximum(h0_ref[...].astype(jnp.float32)
                         + a_ref[...] - b_ref[...] + bh_ref[...], 0.0)
        p_ref[...] = jnp.dot(ht, wh_ref[...],
                             preferred_element_type=jnp.float32)

    return pl.pallas_call(
        body,
        grid=(e // bs,),
        in_specs=[_row_block(bs, h)] * 3 + [_rep_block((1, h)),
                                            _rep_block((h, h))],
        out_specs=_row_block(bs, h),
        out_shape=jax.ShapeDtypeStruct((e, h), jnp.float32),
    )(h0, a, b, bhv, wh)


def _tc_last(h0, a, b, bhv):
    e, h = h0.shape
    bs = 2560

    def body(h0_ref, a_ref, b_ref, bh_ref, o_ref):
        o_ref[...] = jnp.maximum(
            h0_ref[...].astype(jnp.float32)
            + a_ref[...] - b_ref[...] + bh_ref[...], 0.0)

    return pl.pallas_call(
        body,
        grid=(e // bs,),
        in_specs=[_row_block(bs, h)] * 3 + [_rep_block((1, h))],
        out_specs=_row_block(bs, h),
        out_shape=jax.ShapeDtypeStruct((e, h), jnp.float32),
    )(h0, a, b, bhv)


def _tc_merge(p0, p1):
    n, h = p0.shape
    bs = 2000

    def body(a_ref, b_ref, o_ref):
        o_ref[...] = a_ref[...] + b_ref[...]

    return pl.pallas_call(
        body,
        grid=(n // bs,),
        in_specs=[_row_block(bs, h)] * 2,
        out_specs=_row_block(bs, h),
        out_shape=jax.ShapeDtypeStruct((n, h), jnp.float32),
    )(p0, p1)


def _tc_final(p0, p1, x, wox, wom, bo):
    n, h = x.shape
    bs = 2000

    def body(p0_ref, p1_ref, x_ref, wox_ref, wom_ref, bo_ref, o_ref):
        ns = p0_ref[...] + p1_ref[...]
        s = jnp.sum(ns, axis=1, keepdims=True)
        m = jnp.where(s == 0.0, x_ref[...], ns)
        o_ref[...] = jnp.maximum(
            jnp.dot(x_ref[...], wox_ref[...],
                    preferred_element_type=jnp.float32)
            + jnp.dot(m, wom_ref[...], preferred_element_type=jnp.float32)
            + bo_ref[...], 0.0)

    return pl.pallas_call(
        body,
        grid=(n // bs,),
        in_specs=[_row_block(bs, h)] * 3 + [_rep_block((h, h))] * 2
                 + [_rep_block((1, h))],
        out_specs=_row_block(bs, h),
        out_shape=jax.ShapeDtypeStruct((n, h), jnp.float32),
    )(p0, p1, x, wox, wom, bo)


def kernel(x, edge_index, rev_edge_index, edge_attr, W_i, b_i, W_h, b_h,
           W_o, b_o):
    n, df = x.shape
    h = W_h.shape[0]
    e = edge_attr.shape[0]
    src = edge_index[0]
    dst = edge_index[1]
    rev = rev_edge_index
    wi_x, wi_e = W_i[:df], W_i[df:]
    wo_x, wo_m = W_o[:df], W_o[df:]
    bi = b_i.reshape(1, h)
    bhv = b_h.reshape(1, h)
    bo = b_o.reshape(1, h)
    zeros_nd = jnp.zeros((n, h), jnp.float32)

    hx = _tc_matmul(x, wi_x)
    g0 = _sc_gather1h(hx, src, 0, e)
    h0, p = _tc_init(g0, edge_attr, wi_e, bi, W_h)

    for _ in range(_DEPTH - 2):
        parts = _sc_scatter_h(p, dst, 0, zeros_nd)
        ns = _tc_merge(parts[0], parts[1])
        a, b = _sc_gather2h(ns, src, p, rev, 0, e)
        p = _tc_step(h0, a, b, bhv, W_h)

    parts = _sc_scatter_h(p, dst, 0, zeros_nd)
    ns = _tc_merge(parts[0], parts[1])
    a, b = _sc_gather2h(ns, src, p, rev, 0, e)
    h_fin = _tc_last(h0, a, b, bhv)

    parts = _sc_scatter_h(h_fin, dst, 0, zeros_nd)
    return _tc_final(parts[0], parts[1], x, wo_x, wo_m, bo)
